# baseline (speedup 1.0000x reference)
import functools

import jax
import jax.numpy as jnp
from jax import lax
from jax.experimental import pallas as pl
from jax.experimental.pallas import tpu as pltpu
from jax.experimental.pallas import tpu_sc as plsc

VOCAB = 1000000
EMBED = 64
NTOKENS = 200
BATCH = 4096

TOTAL = BATCH * NTOKENS
NUM_WORKERS = 32
PER_WORKER = TOTAL // NUM_WORKERS
CHUNK = NTOKENS
NCHUNKS = PER_WORKER // CHUNK
NGRP = 13
IDXS = 224
ROWS = 208
NITER = NCHUNKS * NGRP + 2

_mesh = plsc.VectorSubcoreMesh(core_axis_name="c", subcore_axis_name="s")


@functools.partial(
    pl.kernel,
    mesh=_mesh,
    out_type=jax.ShapeDtypeStruct((TOTAL, EMBED), jnp.float32),
    scratch_types=[
        pltpu.VMEM((2 * IDXS,), jnp.int32),
        pltpu.VMEM((2 * ROWS, EMBED), jnp.float32),
        pltpu.VMEM((ROWS, EMBED), jnp.float32),
        pltpu.SemaphoreType.DMA,
        pltpu.SemaphoreType.DMA,
        pltpu.SemaphoreType.DMA,
    ],
    compiler_params=pltpu.CompilerParams(use_tc_tiling_on_sc=True),
)
def _embed_sc(tokens_hbm, table_hbm, pos_hbm, out_hbm,
              idx_v, rows_v, pos_v, sem_i, sem_r, sem_o):
    wid = lax.axis_index("s") * 2 + lax.axis_index("c")
    base = wid * PER_WORKER
    last = NCHUNKS - 1
    zeros16 = jnp.zeros((16,), jnp.int32)

    pltpu.sync_copy(pos_hbm, pos_v.at[pl.ds(0, CHUNK)])

    def idx_fetch(c, slot):
        c = jnp.minimum(c, last)
        pltpu.async_copy(tokens_hbm.at[pl.ds(base + c * CHUNK, CHUNK)],
                         idx_v.at[pl.ds(slot * IDXS, CHUNK)], sem_i)

    def idx_wait():
        pltpu.make_async_copy(tokens_hbm.at[pl.ds(0, CHUNK)],
                              idx_v.at[pl.ds(0, CHUNK)], sem_i).wait()

    idx_fetch(0, 0)

    def body(u, carry):
        cf, kf, cs, ks = carry

        @pl.when(cf <= last)
        def _():
            slotf = lax.rem(cf, 2)

            @pl.when(kf == 0)
            def _():
                idx_wait()
                idx_v[pl.ds(slotf * IDXS + CHUNK, 16)] = zeros16

                @pl.when(cf >= 2)
                def _():
                    pltpu.make_async_copy(
                        rows_v.at[pl.ds(0, CHUNK)],
                        out_hbm.at[pl.ds(0, CHUNK)], sem_o).wait()

                idx_fetch(cf + 1, lax.rem(cf + 1, 2))

            gbase = slotf * IDXS + kf * 16
            rbase = slotf * ROWS + kf * 16
            tv = idx_v[pl.ds(gbase, 16)]
            ts = [tv[i] for i in range(16)]
            for i in range(16):
                pltpu.async_copy(table_hbm.at[pl.ds(ts[i], 1)],
                                 rows_v.at[pl.ds(rbase + i, 1)], sem_r)

        @pl.when(ks >= 0)
        def _():
            slots = lax.rem(cs, 2)
            rbase = slots * ROWS + ks * 16
            pltpu.make_async_copy(
                table_hbm.at[pl.ds(0, 16)],
                rows_v.at[pl.ds(rbase, 16)], sem_r).wait()
            for i in range(16):
                r = rbase + i
                p = ks * 16 + i
                for c in range(EMBED // 16):
                    sl = pl.ds(c * 16, 16)
                    rows_v[r, sl] = rows_v[r, sl] + pos_v[p, sl]

            @pl.when(ks == NGRP - 1)
            def _():
                pltpu.async_copy(
                    rows_v.at[pl.ds(slots * ROWS, CHUNK)],
                    out_hbm.at[pl.ds(base + cs * CHUNK, CHUNK)], sem_o)

        kf2 = lax.select(kf == NGRP - 1, 0, kf + 1)
        cf2 = lax.select(kf == NGRP - 1, cf + 1, cf)
        ks2 = lax.select(ks == NGRP - 1, 0, ks + 1)
        cs2 = lax.select(ks == NGRP - 1, cs + 1, cs)
        return cf2, kf2, cs2, ks2

    lax.fori_loop(0, NITER, body, (0, 0, 0, -2))

    idx_wait()
    pltpu.make_async_copy(rows_v.at[pl.ds(0, CHUNK)],
                          out_hbm.at[pl.ds(0, CHUNK)], sem_o).wait()
    pltpu.make_async_copy(rows_v.at[pl.ds(0, CHUNK)],
                          out_hbm.at[pl.ds(0, CHUNK)], sem_o).wait()


def kernel(tokens, input_embedding, position_embedding):
    flat = tokens.reshape(-1).astype(jnp.int32)
    out = _embed_sc(flat, input_embedding, position_embedding)
    return out.reshape(BATCH, NTOKENS, EMBED)
